# baseline (speedup 1.0000x reference)
import jax
import jax.numpy as jnp
from jax import lax
from jax.experimental import pallas as pl
from jax.experimental.pallas import tpu as pltpu
from jax.experimental.pallas import tpu_sc as plsc

N = 320000
D = 128
N_MOLS = 10000
NBLK = N // 128
NW = 32
BPW = NBLK // NW
REM = NBLK - BPW * NW
NPAIRS = BPW // 2
STRIPE = 624
TAIL = N_MOLS - 16 * STRIPE


def _sc_body(h_hbm, batch_hbm, w_hbm, b_hbm, part_hbm,
             hbuf0, hbuf1, idxbuf0, idxbuf1, wvec, bvec, sgbuf, acc,
             sem0, sem1):
    c = lax.axis_index("c")
    s = lax.axis_index("s")
    wid = s * 2 + c

    hbufs = (hbuf0, hbuf1)
    idxbufs = (idxbuf0, idxbuf1)
    sems = (sem0, sem1)

    pltpu.sync_copy(w_hbm, wvec)
    pltpu.sync_copy(b_hbm, bvec)

    def _zero_row(i, _):
        for cc in range(8):
            hbuf0[i, pl.ds(cc * 16, 16)] = jnp.zeros((16,), jnp.float32)
        return 0
    lax.fori_loop(0, 128, _zero_row, 0)
    base = s * STRIPE
    for k in range(4):
        pltpu.sync_copy(hbuf0, acc.at[pl.ds(base + 128 * k, 128)])
    pltpu.sync_copy(hbuf0.at[pl.ds(0, STRIPE - 512)],
                    acc.at[pl.ds(base + 512, STRIPE - 512)])

    @pl.when(s == 15)
    def _zero_tail():
        pltpu.sync_copy(hbuf0.at[pl.ds(0, TAIL)],
                        acc.at[pl.ds(16 * STRIPE, TAIL)])
    plsc.subcore_barrier()

    w_regs = [wvec[pl.ds(cc * 16, 16)] for cc in range(8)]
    b_reg = bvec[...]
    lane = lax.iota(jnp.int32, 16)
    shuf = [jnp.bitwise_xor(lane, k) for k in (8, 4, 2, 1)]
    splats = [jnp.full((16,), k, jnp.int32) for k in range(16)]
    masks = [lane == k for k in range(16)]

    start_blk = wid * BPW + jnp.minimum(wid, REM)
    nblk = BPW + jnp.where(wid < REM, 1, 0)

    def _start_load(blk, p):
        pltpu.make_async_copy(h_hbm.at[pl.ds(blk * 128, 128)],
                              hbufs[p], sems[p]).start()
        pltpu.make_async_copy(batch_hbm.at[pl.ds(blk, 1)],
                              idxbufs[p], sems[p]).start()

    def _wait_load(p):
        pltpu.make_async_copy(h_hbm.at[pl.ds(0, 128)],
                              hbufs[p], sems[p]).wait()
        pltpu.make_async_copy(batch_hbm.at[pl.ds(0, 1)],
                              idxbufs[p], sems[p]).wait()

    def _process(hb, ib):
        @plsc.parallel_loop(0, 8)
        def _gate(g):
            r0 = g * 16
            dots = jnp.zeros((16,), jnp.float32)
            for k in range(16):
                h = [hb[r0 + k, pl.ds(cc * 16, 16)] for cc in range(8)]
                m0 = h[0] * w_regs[0]
                m1 = h[1] * w_regs[1]
                m2 = h[2] * w_regs[2]
                m3 = h[3] * w_regs[3]
                m4 = h[4] * w_regs[4]
                m5 = h[5] * w_regs[5]
                m6 = h[6] * w_regs[6]
                m7 = h[7] * w_regs[7]
                t = ((m0 + m1) + (m2 + m3)) + ((m4 + m5) + (m6 + m7))
                for sh in shuf:
                    t = t + t.at[sh].get(mode="promise_in_bounds")
                dots = jnp.where(masks[k], t, dots)
            dv = dots + b_reg
            sgbuf[pl.ds(g * 16, 16)] = 1.0 / (1.0 + jnp.exp(-dv))

        @plsc.parallel_loop(0, 8)
        def _scale(g):
            r0 = g * 16
            sgv = sgbuf[pl.ds(g * 16, 16)]
            for k in range(16):
                sk = sgv.at[splats[k]].get(mode="promise_in_bounds")
                for cc in range(8):
                    sl = pl.ds(cc * 16, 16)
                    hb[r0 + k, sl] = hb[r0 + k, sl] * sk
        pltpu.sync_copy(hb, acc.at[ib.at[0]], add=True)

    _start_load(start_blk, 0)
    _start_load(start_blk + 1, 1)

    def _pair(jj, _):
        for p in (0, 1):
            jl = 2 * jj + p
            _wait_load(p)
            _process(hbufs[p], idxbufs[p])

            @pl.when(jl + 2 < nblk)
            def _next():
                _start_load(start_blk + jl + 2, p)
        return 0
    lax.fori_loop(0, NPAIRS, _pair, 0)

    @pl.when(wid < REM)
    def _tail_block():
        _wait_load(0)
        _process(hbuf0, idxbuf0)

    plsc.subcore_barrier()
    pltpu.sync_copy(acc.at[pl.ds(base, STRIPE)],
                    part_hbm.at[c, pl.ds(base, STRIPE)])

    @pl.when(s == 15)
    def _flush_tail():
        pltpu.sync_copy(acc.at[pl.ds(16 * STRIPE, TAIL)],
                        part_hbm.at[c, pl.ds(16 * STRIPE, TAIL)])


def _merge_body(p_ref, o_ref):
    o_ref[...] = p_ref[0] + p_ref[1]


def kernel(H, batch, W, b):
    batch2 = batch.astype(jnp.int32).reshape(NBLK, 128)
    w128 = W.reshape(D)
    b16 = jnp.broadcast_to(b.reshape(1), (16,)).astype(jnp.float32)

    sc = pl.kernel(
        _sc_body,
        out_type=jax.ShapeDtypeStruct((2, N_MOLS, D), jnp.float32),
        mesh=plsc.VectorSubcoreMesh(core_axis_name="c", subcore_axis_name="s"),
        scratch_types=[
            pltpu.VMEM((128, D), jnp.float32),
            pltpu.VMEM((128, D), jnp.float32),
            pltpu.VMEM((1, 128), jnp.int32),
            pltpu.VMEM((1, 128), jnp.int32),
            pltpu.VMEM((D,), jnp.float32),
            pltpu.VMEM((16,), jnp.float32),
            pltpu.VMEM((128,), jnp.float32),
            pltpu.VMEM_SHARED((N_MOLS, D), jnp.float32),
            pltpu.SemaphoreType.DMA,
            pltpu.SemaphoreType.DMA,
        ],
    )
    partials = sc(H, batch2, w128, b16)

    merge = pl.pallas_call(
        _merge_body,
        out_shape=jax.ShapeDtypeStruct((N_MOLS, D), jnp.float32),
        grid=(N_MOLS // 400,),
        in_specs=[pl.BlockSpec((2, 400, D), lambda i: (0, i, 0))],
        out_specs=pl.BlockSpec((400, D), lambda i: (i, 0)),
    )
    return merge(partials)


# speedup vs baseline: 1.8126x; 1.8126x over previous
import jax
import jax.numpy as jnp
from jax import lax
from jax.experimental import pallas as pl
from jax.experimental.pallas import tpu as pltpu
from jax.experimental.pallas import tpu_sc as plsc

N = 320000
D = 128
N_MOLS = 10000
NBLK = N // 128
NW = 32
BPW = NBLK // NW
REM = NBLK - BPW * NW
NTRIP = BPW // 3
STRIPE = 624
TAIL = N_MOLS - 16 * STRIPE


def _sc_body(h_hbm, batch_hbm, w_hbm, b_hbm, part_hbm,
             hbuf0, hbuf1, hbuf2, idxbuf0, idxbuf1, idxbuf2,
             sidx0, sidx1, sidx2, wvec, bvec, sgbuf, acc,
             sem0, sem1, sem2, ssem0, ssem1, ssem2):
    c = lax.axis_index("c")
    s = lax.axis_index("s")
    wid = s * 2 + c

    hbufs = (hbuf0, hbuf1, hbuf2)
    idxbufs = (idxbuf0, idxbuf1, idxbuf2)
    sidxs = (sidx0, sidx1, sidx2)
    sems = (sem0, sem1, sem2)
    ssems = (ssem0, ssem1, ssem2)

    start_blk = wid * BPW + jnp.minimum(wid, REM)
    nblk = BPW + jnp.where(wid < REM, 1, 0)

    def _start_load(blk, p):
        pltpu.make_async_copy(h_hbm.at[pl.ds(blk * 128, 128)],
                              hbufs[p], sems[p]).start()
        pltpu.make_async_copy(batch_hbm.at[pl.ds(blk, 1)],
                              idxbufs[p], sems[p]).start()

    def _wait_load(p):
        pltpu.make_async_copy(h_hbm.at[pl.ds(0, 128)],
                              hbufs[p], sems[p]).wait()
        pltpu.make_async_copy(batch_hbm.at[pl.ds(0, 1)],
                              idxbufs[p], sems[p]).wait()

    def _wait_scatter(p):
        pltpu.make_async_copy(hbufs[p], acc.at[sidxs[p].at[0]],
                              ssems[p]).wait()

    _start_load(start_blk, 0)

    pltpu.sync_copy(w_hbm, wvec)
    pltpu.sync_copy(b_hbm, bvec)

    def _zero_row(i, _):
        for cc in range(8):
            hbuf1[i, pl.ds(cc * 16, 16)] = jnp.zeros((16,), jnp.float32)
        return 0
    lax.fori_loop(0, 128, _zero_row, 0)
    base = s * STRIPE
    for k in range(4):
        pltpu.sync_copy(hbuf1, acc.at[pl.ds(base + 128 * k, 128)])
    pltpu.sync_copy(hbuf1.at[pl.ds(0, STRIPE - 512)],
                    acc.at[pl.ds(base + 512, STRIPE - 512)])

    @pl.when(s == 15)
    def _zero_tail():
        pltpu.sync_copy(hbuf1.at[pl.ds(0, TAIL)],
                        acc.at[pl.ds(16 * STRIPE, TAIL)])
    plsc.subcore_barrier()

    w_regs = [wvec[pl.ds(cc * 16, 16)] for cc in range(8)]
    b_reg = bvec[...]
    lane = lax.iota(jnp.int32, 16)
    shuf = [jnp.bitwise_xor(lane, k) for k in (8, 4, 2, 1)]
    splats = [jnp.full((16,), k, jnp.int32) for k in range(16)]
    masks = [lane == k for k in range(16)]

    def _compute(hb):
        def _gate(g, _):
            r0 = g * 16
            dots = jnp.zeros((16,), jnp.float32)
            for k in range(16):
                h = [hb[r0 + k, pl.ds(cc * 16, 16)] for cc in range(8)]
                m0 = h[0] * w_regs[0]
                m1 = h[1] * w_regs[1]
                m2 = h[2] * w_regs[2]
                m3 = h[3] * w_regs[3]
                m4 = h[4] * w_regs[4]
                m5 = h[5] * w_regs[5]
                m6 = h[6] * w_regs[6]
                m7 = h[7] * w_regs[7]
                t = ((m0 + m1) + (m2 + m3)) + ((m4 + m5) + (m6 + m7))
                for sh in shuf:
                    t = t + t.at[sh].get(mode="promise_in_bounds")
                dots = jnp.where(masks[k], t, dots)
            dv = dots + b_reg
            sgbuf[pl.ds(g * 16, 16)] = 1.0 / (1.0 + jnp.exp(-dv))
            return 0
        lax.fori_loop(0, 8, _gate, 0)

        def _scale(g, _):
            r0 = g * 16
            sgv = sgbuf[pl.ds(g * 16, 16)]
            for k in range(16):
                sk = sgv.at[splats[k]].get(mode="promise_in_bounds")
                for cc in range(8):
                    sl = pl.ds(cc * 16, 16)
                    hb[r0 + k, sl] = hb[r0 + k, sl] * sk
            return 0
        lax.fori_loop(0, 8, _scale, 0)

    def _step(jl, p):
        @pl.when(jl >= 2)
        def _free_prev():
            _wait_scatter((p + 1) % 3)

        @pl.when(jl + 1 < nblk)
        def _prefetch():
            _start_load(start_blk + jl + 1, (p + 1) % 3)

        _wait_load(p)
        _compute(hbufs[p])
        for cc in range(8):
            sl = pl.ds(cc * 16, 16)
            sidxs[p][0, sl] = idxbufs[p][0, sl]
        pltpu.async_copy(hbufs[p], acc.at[sidxs[p].at[0]], ssems[p], add=True)

    def _trip(jj, _):
        for p in (0, 1, 2):
            _step(3 * jj + p, p)
        return 0
    lax.fori_loop(0, NTRIP, _trip, 0)

    @pl.when(wid < REM)
    def _tail_block():
        _step(BPW, 0)

    last = nblk - 1
    for p in (0, 1, 2):
        @pl.when((lax.rem(last, 3) == p) | (lax.rem(last - 1, 3) == p))
        def _drain():
            _wait_scatter(p)

    plsc.subcore_barrier()
    pltpu.sync_copy(acc.at[pl.ds(base, STRIPE)],
                    part_hbm.at[c, pl.ds(base, STRIPE)])

    @pl.when(s == 15)
    def _flush_tail():
        pltpu.sync_copy(acc.at[pl.ds(16 * STRIPE, TAIL)],
                        part_hbm.at[c, pl.ds(16 * STRIPE, TAIL)])


def _merge_body(p_ref, o_ref):
    o_ref[...] = p_ref[0] + p_ref[1]


def kernel(H, batch, W, b):
    batch2 = batch.astype(jnp.int32).reshape(NBLK, 128)
    w128 = W.reshape(D)
    b16 = jnp.broadcast_to(b.reshape(1), (16,)).astype(jnp.float32)

    sc = pl.kernel(
        _sc_body,
        out_type=jax.ShapeDtypeStruct((2, N_MOLS, D), jnp.float32),
        mesh=plsc.VectorSubcoreMesh(core_axis_name="c", subcore_axis_name="s"),
        scratch_types=[
            pltpu.VMEM((128, D), jnp.float32),
            pltpu.VMEM((128, D), jnp.float32),
            pltpu.VMEM((128, D), jnp.float32),
            pltpu.VMEM((1, 128), jnp.int32),
            pltpu.VMEM((1, 128), jnp.int32),
            pltpu.VMEM((1, 128), jnp.int32),
            pltpu.VMEM((1, 128), jnp.int32),
            pltpu.VMEM((1, 128), jnp.int32),
            pltpu.VMEM((1, 128), jnp.int32),
            pltpu.VMEM((D,), jnp.float32),
            pltpu.VMEM((16,), jnp.float32),
            pltpu.VMEM((128,), jnp.float32),
            pltpu.VMEM_SHARED((N_MOLS, D), jnp.float32),
            pltpu.SemaphoreType.DMA,
            pltpu.SemaphoreType.DMA,
            pltpu.SemaphoreType.DMA,
            pltpu.SemaphoreType.DMA,
            pltpu.SemaphoreType.DMA,
            pltpu.SemaphoreType.DMA,
        ],
    )
    partials = sc(H, batch2, w128, b16)

    merge = pl.pallas_call(
        _merge_body,
        out_shape=jax.ShapeDtypeStruct((N_MOLS, D), jnp.float32),
        grid=(N_MOLS // 400,),
        in_specs=[pl.BlockSpec((2, 400, D), lambda i: (0, i, 0))],
        out_specs=pl.BlockSpec((400, D), lambda i: (i, 0)),
    )
    return merge(partials)
